# baseline (speedup 1.0000x reference)
import functools

import jax
import jax.numpy as jnp
from jax import lax
from jax.experimental import pallas as pl
from jax.experimental.pallas import tpu as pltpu
from jax.experimental.pallas import tpu_sc as plsc

B, H, S, D, T = 8, 16, 2048, 128, 16
BHS = B * H * S

NC, NS = 1, 8
NW = NC * NS
ROWS = B * H * T
RPW = ROWS // NW
GPW = RPW // T

_mesh = plsc.VectorSubcoreMesh(core_axis_name="c", subcore_axis_name="s", num_cores=1, num_subcores=8)


@functools.partial(
    pl.kernel,
    out_type=(),
    mesh=_mesh,
    scratch_types=[
        pltpu.VMEM((T,), jnp.int32),
        pltpu.VMEM((RPW,), jnp.int32),
        pltpu.VMEM((RPW, D), jnp.float32),
        pltpu.VMEM((RPW, D), jnp.float32),
        pltpu.SemaphoreType.DMA,
        pltpu.SemaphoreType.DMA,
    ],
)
def _scatter_update(ck_ref, cv_ref, nk_hbm, nv_hbm, pos_hbm,
                    pos_v, idx_v, krows_v, vrows_v, semk, semv):
    wid = lax.axis_index("s") * NC + lax.axis_index("c")
    base = wid * RPW
    cpk_in = pltpu.async_copy(nk_hbm.at[pl.ds(base, RPW)], krows_v, semk)
    cpv_in = pltpu.async_copy(nv_hbm.at[pl.ds(base, RPW)], vrows_v, semv)
    pltpu.sync_copy(pos_hbm, pos_v)
    pos = pos_v[...]
    for g in range(GPW):
        bh = wid * GPW + g
        idx_v[pl.ds(g * T, T)] = pos + bh * S
    cpk_in.wait()
    cpv_in.wait()
    cpk = pltpu.async_copy(krows_v, ck_ref.at[idx_v], semk)
    cpv = pltpu.async_copy(vrows_v, cv_ref.at[idx_v], semv)
    cpk.wait()
    cpv.wait()


def kernel(cache_k, cache_v, new_k, new_v, position_ids):
    pos = position_ids.astype(jnp.int32)
    ck = jax.new_ref(cache_k.reshape(BHS, D))
    cv = jax.new_ref(cache_v.reshape(BHS, D))
    _scatter_update(ck, cv,
                    new_k.reshape(ROWS, D),
                    new_v.reshape(ROWS, D),
                    pos)
    return (ck[...].reshape(B, H, S, D), cv[...].reshape(B, H, S, D))


# speedup vs baseline: 1.0141x; 1.0141x over previous
import functools

import jax
import jax.numpy as jnp
from jax import lax
from jax.experimental import pallas as pl
from jax.experimental.pallas import tpu as pltpu
from jax.experimental.pallas import tpu_sc as plsc

B, H, S, D, T = 8, 16, 2048, 128, 16
BHS = B * H * S

NC, NS = 1, 16
NW = NC * NS
ROWS = B * H * T
RPW = ROWS // NW
GPW = RPW // T

_mesh = plsc.VectorSubcoreMesh(core_axis_name="c", subcore_axis_name="s",
                               num_cores=NC)


@functools.partial(
    pl.kernel,
    out_type=(),
    mesh=_mesh,
    scratch_types=[
        pltpu.VMEM((T,), jnp.int32),
        pltpu.VMEM((RPW,), jnp.int32),
        pltpu.VMEM((RPW, D), jnp.float32),
        pltpu.VMEM((RPW, D), jnp.float32),
        pltpu.SemaphoreType.DMA,
        pltpu.SemaphoreType.DMA,
    ],
)
def _scatter_update(ck_ref, cv_ref, nk_hbm, nv_hbm, pos_hbm,
                    pos_v, idx_v, krows_v, vrows_v, semk, semv):
    wid = lax.axis_index("s") * NC + lax.axis_index("c")
    base = wid * RPW
    cpk_in = pltpu.async_copy(nk_hbm.at[pl.ds(base, RPW)], krows_v, semk)
    cpv_in = pltpu.async_copy(nv_hbm.at[pl.ds(base, RPW)], vrows_v, semv)
    pltpu.sync_copy(pos_hbm, pos_v)
    pos = pos_v[...]
    for g in range(GPW):
        bh = wid * GPW + g
        idx_v[pl.ds(g * T, T)] = pos + bh * S
    cpk_in.wait()
    cpv_in.wait()
    cpk = pltpu.async_copy(krows_v, ck_ref.at[idx_v], semk)
    cpv = pltpu.async_copy(vrows_v, cv_ref.at[idx_v], semv)
    cpk.wait()
    cpv.wait()


def kernel(cache_k, cache_v, new_k, new_v, position_ids):
    pos = position_ids.astype(jnp.int32)
    ck = jax.new_ref(cache_k.reshape(BHS, D))
    cv = jax.new_ref(cache_v.reshape(BHS, D))
    _scatter_update(ck, cv,
                    new_k.reshape(ROWS, D),
                    new_v.reshape(ROWS, D),
                    pos)
    return (ck[...].reshape(B, H, S, D), cv[...].reshape(B, H, S, D))
